# baseline (speedup 1.0000x reference)
import functools

import numpy as np
import jax
import jax.numpy as jnp
from jax import lax
from jax.experimental import pallas as pl
from jax.experimental.pallas import tpu as pltpu
from jax.experimental.pallas import tpu_sc as plsc

_NUMANGLE = 100
_NUMRHO = 100
_B, _C, _H, _W = 1, 128, 100, 100
_P = _H * _W

_N_TC = 56
_ATC = 8
_N_SC = _NUMANGLE - _N_TC

_NSC = 2
_NSUB = 16
_CHUNK = 128
_NCH = 5
_PSUB = _NCH * _CHUNK
_PPAD = _NSUB * _PSUB
_APC = _N_SC // _NSC
_ROWS = _APC * _NUMRHO
_RSUB = ((_ROWS + _NSUB - 1) // _NSUB + 7) // 8 * 8
_G = 11
_RPAD = _NSUB * _RSUB


def _make_rho_table():
    irho = int(np.sqrt(_H * _H + _W * _W) + 1) / float(_NUMRHO - 1)
    itheta = np.pi / _NUMANGLE
    angles = np.arange(_NUMANGLE) * itheta
    tab_cos = (np.cos(angles) / irho).astype(np.float32)
    tab_sin = (np.sin(angles) / irho).astype(np.float32)
    ys, xs = np.meshgrid(np.arange(_H), np.arange(_W), indexing="ij")
    xx = (xs - _W // 2).astype(np.float32)
    yy = (ys - _H // 2).astype(np.float32)
    r = np.round(xx[None] * tab_cos[:, None, None] + yy[None] * tab_sin[:, None, None])
    r = r.astype(np.int32) + _NUMRHO // 2
    return r.reshape(_NUMANGLE, _P)


_RTAB = _make_rho_table()


def _make_idx_table():
    r = _RTAB[_N_TC:]
    local = (np.arange(_N_SC) % _APC)[:, None] * _NUMRHO + r
    pad = np.broadcast_to(
        (np.arange(_PPAD - _P, dtype=np.int32) * 7) % _ROWS, (_N_SC, _PPAD - _P)
    )
    t = np.concatenate([local.astype(np.int32), pad], axis=1)
    t = t.reshape(_N_SC, _NSUB, _NCH, _CHUNK)
    t = t.transpose(1, 0, 2, 3).reshape(_NSUB, _NSC, _APC, _NCH, _CHUNK)
    return np.ascontiguousarray(t)


_IDXTAB = _make_idx_table()


def _sc_body(vt_hbm, idx_hbm, out_hbm, v_vmem, idx_vmem, acc, sem):
    c = lax.axis_index("c")
    s = lax.axis_index("s")

    def zrow(i, carry):
        for j in range(_C // 16):
            v_vmem[i, pl.ds(j * 16, 16)] = jnp.zeros((16,), jnp.float32)
        return carry

    lax.fori_loop(0, _RSUB, zrow, 0)
    pltpu.sync_copy(v_vmem.at[pl.ds(0, _RSUB)], acc.at[pl.ds(s * _RSUB, _RSUB)])

    pltpu.sync_copy(vt_hbm.at[pl.ds(s * _PSUB, _PSUB)], v_vmem)
    pltpu.sync_copy(idx_hbm.at[s, c], idx_vmem)
    plsc.subcore_barrier()

    def group(g, carry):
        handles = []
        for e in range(_G):
            for j in range(_NCH):
                handles.append(
                    pltpu.async_copy(
                        v_vmem.at[pl.ds(j * _CHUNK, _CHUNK)],
                        acc.at[idx_vmem.at[g * _G + e, j]],
                        sem,
                        add=True,
                    )
                )
        for h in handles:
            h.wait()
        return carry

    lax.fori_loop(0, _APC // _G, group, 0)
    plsc.subcore_barrier()

    pltpu.sync_copy(
        acc.at[pl.ds(s * _RSUB, _RSUB)],
        out_hbm.at[c, pl.ds(s * _RSUB, _RSUB)],
    )


_dht_sc = functools.partial(
    pl.kernel,
    out_type=jax.ShapeDtypeStruct((_NSC, _RPAD, _C), jnp.float32),
    mesh=plsc.VectorSubcoreMesh(core_axis_name="c", subcore_axis_name="s"),
    scratch_types=[
        pltpu.VMEM((_PSUB, _C), jnp.float32),
        pltpu.VMEM((_APC, _NCH, _CHUNK), jnp.int32),
        pltpu.VMEM_SHARED((_RPAD, _C), jnp.float32),
        pltpu.SemaphoreType.DMA,
    ],
)(_sc_body)


def _tc_body(r_ref, x_ref, out_ref, v_ref):
    a = pl.program_id(0)

    @pl.when(a == 0)
    def _():
        v_ref[...] = jnp.maximum(x_ref[...], 0.0).astype(jnp.bfloat16)

    one = jnp.ones((), jnp.bfloat16)
    zero = jnp.zeros((), jnp.bfloat16)
    iota2 = jax.lax.broadcasted_iota(jnp.int16, (_P, 2 * _NUMRHO), 1)
    rho2 = jnp.where(iota2 < _NUMRHO, iota2, iota2 - _NUMRHO)
    for i in range(_ATC // 2):
        r0 = r_ref[2 * i, 0, :]
        r1 = r_ref[2 * i + 1, 0, :]
        rsel = jnp.where(iota2 < _NUMRHO, r0[:, None], r1[:, None])
        onehot2 = jnp.where(rsel == rho2, one, zero)
        res = jnp.dot(v_ref[...], onehot2, preferred_element_type=jnp.float32)
        out_ref[:, 2 * i, :] = res[:, :_NUMRHO]
        out_ref[:, 2 * i + 1, :] = res[:, _NUMRHO:]


def _tc_call(r3, v2):
    return pl.pallas_call(
        _tc_body,
        grid=(_N_TC // _ATC,),
        in_specs=[
            pl.BlockSpec((_ATC, 1, _P), lambda a: (a, 0, 0)),
            pl.BlockSpec((_C, _P), lambda a: (0, 0)),
        ],
        out_specs=pl.BlockSpec((_C, _ATC, _NUMRHO), lambda a: (0, a, 0)),
        out_shape=jax.ShapeDtypeStruct((_C, _N_TC, _NUMRHO), jnp.float32),
        scratch_shapes=[pltpu.VMEM((_C, _P), jnp.bfloat16)],
    )(r3, v2)


def kernel(x):
    v2 = x.reshape(_C, _P)

    vt = jnp.pad(jnp.maximum(v2, 0.0).T, ((0, _PPAD - _P), (0, 0)))
    out_sc = _dht_sc(vt, jnp.asarray(_IDXTAB))

    r3 = jnp.asarray(_RTAB[:_N_TC].astype(np.int16))[:, None, :]
    acc_tc = _tc_call(r3, v2)

    acc_sc = (
        out_sc[:, :_ROWS]
        .reshape(_N_SC * _NUMRHO, _C)
        .T.reshape(_C, _N_SC, _NUMRHO)
    )
    out = jnp.concatenate([acc_tc, acc_sc], axis=1)
    return out.reshape(_B, _C, _NUMANGLE, _NUMRHO)


# speedup vs baseline: 1.0041x; 1.0041x over previous
import functools

import numpy as np
import jax
import jax.numpy as jnp
from jax import lax
from jax.experimental import pallas as pl
from jax.experimental.pallas import tpu as pltpu
from jax.experimental.pallas import tpu_sc as plsc

_NUMANGLE = 100
_NUMRHO = 100
_B, _C, _H, _W = 1, 128, 100, 100
_P = _H * _W

_N_TC = 56
_ATC = 8
_N_SC = _NUMANGLE - _N_TC

_NSC = 2
_NSUB = 16
_CHUNK = 128
_NCH = 5
_PSUB = _NCH * _CHUNK
_PPAD = _NSUB * _PSUB
_APC = _N_SC // _NSC
_ROWS = _APC * _NUMRHO
_RSUB = ((_ROWS + _NSUB - 1) // _NSUB + 7) // 8 * 8
_G = 11
_RPAD = _NSUB * _RSUB


def _make_rho_table():
    irho = int(np.sqrt(_H * _H + _W * _W) + 1) / float(_NUMRHO - 1)
    itheta = np.pi / _NUMANGLE
    angles = np.arange(_NUMANGLE) * itheta
    tab_cos = (np.cos(angles) / irho).astype(np.float32)
    tab_sin = (np.sin(angles) / irho).astype(np.float32)
    ys, xs = np.meshgrid(np.arange(_H), np.arange(_W), indexing="ij")
    xx = (xs - _W // 2).astype(np.float32)
    yy = (ys - _H // 2).astype(np.float32)
    r = np.round(xx[None] * tab_cos[:, None, None] + yy[None] * tab_sin[:, None, None])
    r = r.astype(np.int32) + _NUMRHO // 2
    return r.reshape(_NUMANGLE, _P)


_RTAB = _make_rho_table()


def _make_idx_table():
    r = _RTAB[_N_TC:]
    local = (np.arange(_N_SC) % _APC)[:, None] * _NUMRHO + r
    pad = np.broadcast_to(
        (np.arange(_PPAD - _P, dtype=np.int32) * 7) % _ROWS, (_N_SC, _PPAD - _P)
    )
    t = np.concatenate([local.astype(np.int32), pad], axis=1)
    t = t.reshape(_N_SC, _NSUB, _NCH, _CHUNK)
    t = t.transpose(1, 0, 2, 3).reshape(_NSUB, _NSC, _APC, _NCH, _CHUNK)
    return np.ascontiguousarray(t)


_IDXTAB = _make_idx_table()


def _sc_body(vt_hbm, idx_hbm, out_hbm, v_vmem, idx_vmem, acc, sem):
    c = lax.axis_index("c")
    s = lax.axis_index("s")

    def zrow(i, carry):
        for j in range(_C // 16):
            v_vmem[i, pl.ds(j * 16, 16)] = jnp.zeros((16,), jnp.float32)
        return carry

    lax.fori_loop(0, _RSUB, zrow, 0)
    pltpu.sync_copy(v_vmem.at[pl.ds(0, _RSUB)], acc.at[pl.ds(s * _RSUB, _RSUB)])

    pltpu.sync_copy(vt_hbm.at[pl.ds(s * _PSUB, _PSUB)], v_vmem)
    pltpu.sync_copy(idx_hbm.at[s, c], idx_vmem)
    plsc.subcore_barrier()

    def group(g, carry):
        handles = []
        for e in range(_G):
            for j in range(_NCH):
                handles.append(
                    pltpu.async_copy(
                        v_vmem.at[pl.ds(j * _CHUNK, _CHUNK)],
                        acc.at[idx_vmem.at[g * _G + e, j]],
                        sem,
                        add=True,
                    )
                )
        for h in handles:
            h.wait()
        return carry

    lax.fori_loop(0, _APC // _G, group, 0)
    plsc.subcore_barrier()

    pltpu.sync_copy(
        acc.at[pl.ds(s * _RSUB, _RSUB)],
        out_hbm.at[c, pl.ds(s * _RSUB, _RSUB)],
    )


_dht_sc = functools.partial(
    pl.kernel,
    out_type=jax.ShapeDtypeStruct((_NSC, _RPAD, _C), jnp.float32),
    mesh=plsc.VectorSubcoreMesh(core_axis_name="c", subcore_axis_name="s"),
    scratch_types=[
        pltpu.VMEM((_PSUB, _C), jnp.float32),
        pltpu.VMEM((_APC, _NCH, _CHUNK), jnp.int32),
        pltpu.VMEM_SHARED((_RPAD, _C), jnp.float32),
        pltpu.SemaphoreType.DMA,
    ],
)(_sc_body)


def _tc_body(r_ref, x_ref, out_ref, v_ref):
    a = pl.program_id(0)

    @pl.when(a == 0)
    def _():
        v_ref[...] = jnp.maximum(x_ref[...], 0.0).astype(jnp.bfloat16)

    one = jnp.ones((), jnp.bfloat16)
    zero = jnp.zeros((), jnp.bfloat16)
    iota = jax.lax.broadcasted_iota(jnp.int16, (_P, _NUMRHO), 1)
    for i in range(_ATC):
        r = r_ref[i, 0, :]
        onehot = jnp.where(r[:, None] == iota, one, zero)
        out_ref[:, i, :] = jnp.dot(
            v_ref[...], onehot, preferred_element_type=jnp.float32
        )


def _tc_call(r3, v2):
    return pl.pallas_call(
        _tc_body,
        grid=(_N_TC // _ATC,),
        in_specs=[
            pl.BlockSpec((_ATC, 1, _P), lambda a: (a, 0, 0)),
            pl.BlockSpec((_C, _P), lambda a: (0, 0)),
        ],
        out_specs=pl.BlockSpec((_C, _ATC, _NUMRHO), lambda a: (0, a, 0)),
        out_shape=jax.ShapeDtypeStruct((_C, _N_TC, _NUMRHO), jnp.float32),
        scratch_shapes=[pltpu.VMEM((_C, _P), jnp.bfloat16)],
    )(r3, v2)


def kernel(x):
    v2 = x.reshape(_C, _P)

    vt = jnp.pad(jnp.maximum(v2, 0.0).T, ((0, _PPAD - _P), (0, 0)))
    out_sc = _dht_sc(vt, jnp.asarray(_IDXTAB))

    r3 = jnp.asarray(_RTAB[:_N_TC].astype(np.int16))[:, None, :]
    acc_tc = _tc_call(r3, v2)

    acc_sc = (
        out_sc[:, :_ROWS]
        .reshape(_N_SC * _NUMRHO, _C)
        .T.reshape(_C, _N_SC, _NUMRHO)
    )
    out = jnp.concatenate([acc_tc, acc_sc], axis=1)
    return out.reshape(_B, _C, _NUMANGLE, _NUMRHO)


# speedup vs baseline: 1.3394x; 1.3339x over previous
import functools

import numpy as np
import jax
import jax.numpy as jnp
from jax import lax
from jax.experimental import pallas as pl
from jax.experimental.pallas import tpu as pltpu
from jax.experimental.pallas import tpu_sc as plsc

_NUMANGLE = 100
_NUMRHO = 100
_B, _C, _H, _W = 1, 128, 100, 100
_P = _H * _W

_N_TC = 72
_ATC = 8
_N_SC = _NUMANGLE - _N_TC

_NSC = 2
_NSUB = 16
_CHUNK = 128
_NCH = 5
_PSUB = _NCH * _CHUNK
_PPAD = _NSUB * _PSUB
_APC = _N_SC // _NSC
_RSTRIDE = 104
_ROWS = _APC * _RSTRIDE
_RSUB = ((_ROWS + _NSUB - 1) // _NSUB + 7) // 8 * 8
_RPAD = _NSUB * _RSUB
_G = _APC


def _make_rho_table():
    irho = int(np.sqrt(_H * _H + _W * _W) + 1) / float(_NUMRHO - 1)
    itheta = np.pi / _NUMANGLE
    angles = np.arange(_NUMANGLE) * itheta
    tab_cos = (np.cos(angles) / irho).astype(np.float32)
    tab_sin = (np.sin(angles) / irho).astype(np.float32)
    ys, xs = np.meshgrid(np.arange(_H), np.arange(_W), indexing="ij")
    xx = (xs - _W // 2).astype(np.float32)
    yy = (ys - _H // 2).astype(np.float32)
    r = np.round(xx[None] * tab_cos[:, None, None] + yy[None] * tab_sin[:, None, None])
    r = r.astype(np.int32) + _NUMRHO // 2
    return r.reshape(_NUMANGLE, _P)


_RTAB = _make_rho_table()


def _make_idx_table():
    r = _RTAB[_N_TC:]
    local = (np.arange(_N_SC) % _APC)[:, None] * _RSTRIDE + r
    pad = np.broadcast_to(
        (np.arange(_PPAD - _P, dtype=np.int32) * 7) % _ROWS, (_N_SC, _PPAD - _P)
    )
    t = np.concatenate([local.astype(np.int32), pad], axis=1)
    t = t.reshape(_N_SC, _NSUB, _NCH, _CHUNK)
    t = t.transpose(1, 0, 2, 3).reshape(_NSUB, _NSC, _APC, _NCH, _CHUNK)
    return np.ascontiguousarray(t)


_IDXTAB = _make_idx_table()


def _sc_body(vt_hbm, idx_hbm, out_hbm, v_vmem, idx_vmem, acc, sem):
    c = lax.axis_index("c")
    s = lax.axis_index("s")

    def zrow(i, carry):
        for j in range(_C // 16):
            v_vmem[i, pl.ds(j * 16, 16)] = jnp.zeros((16,), jnp.float32)
        return carry

    lax.fori_loop(0, _RSUB, zrow, 0)
    pltpu.sync_copy(v_vmem.at[pl.ds(0, _RSUB)], acc.at[pl.ds(s * _RSUB, _RSUB)])

    pltpu.sync_copy(vt_hbm.at[pl.ds(s * _PSUB, _PSUB)], v_vmem)
    pltpu.sync_copy(idx_hbm.at[s, c], idx_vmem)
    plsc.subcore_barrier()

    def group(g, carry):
        handles = []
        for e in range(_G):
            for j in range(_NCH):
                handles.append(
                    pltpu.async_copy(
                        v_vmem.at[pl.ds(j * _CHUNK, _CHUNK)],
                        acc.at[idx_vmem.at[g * _G + e, j]],
                        sem,
                        add=True,
                    )
                )
        for h in handles:
            h.wait()
        return carry

    lax.fori_loop(0, _APC // _G, group, 0)
    plsc.subcore_barrier()

    pltpu.sync_copy(
        acc.at[pl.ds(s * _RSUB, _RSUB)],
        out_hbm.at[c, pl.ds(s * _RSUB, _RSUB)],
    )


_dht_sc = functools.partial(
    pl.kernel,
    out_type=jax.ShapeDtypeStruct((_NSC, _RPAD, _C), jnp.float32),
    mesh=plsc.VectorSubcoreMesh(core_axis_name="c", subcore_axis_name="s"),
    scratch_types=[
        pltpu.VMEM((_PSUB, _C), jnp.float32),
        pltpu.VMEM((_APC, _NCH, _CHUNK), jnp.int32),
        pltpu.VMEM_SHARED((_RPAD, _C), jnp.float32),
        pltpu.SemaphoreType.DMA,
    ],
)(_sc_body)


def _tc_body(r_ref, x_ref, out_ref, v_ref):
    a = pl.program_id(0)

    @pl.when(a == 0)
    def _():
        v_ref[...] = jnp.maximum(x_ref[...], 0.0).astype(jnp.bfloat16)

    one = jnp.ones((), jnp.bfloat16)
    zero = jnp.zeros((), jnp.bfloat16)
    iota = jax.lax.broadcasted_iota(jnp.int16, (_P, _NUMRHO), 1)
    for i in range(_ATC):
        r = r_ref[i, 0, :]
        onehot = jnp.where(r[:, None] == iota, one, zero)
        out_ref[:, i, :] = jnp.dot(
            v_ref[...], onehot, preferred_element_type=jnp.float32
        )


def _tc_call(r3, v2):
    return pl.pallas_call(
        _tc_body,
        grid=(_N_TC // _ATC,),
        in_specs=[
            pl.BlockSpec((_ATC, 1, _P), lambda a: (a, 0, 0)),
            pl.BlockSpec((_C, _P), lambda a: (0, 0)),
        ],
        out_specs=pl.BlockSpec((_C, _ATC, _NUMRHO), lambda a: (0, a, 0)),
        out_shape=jax.ShapeDtypeStruct((_C, _N_TC, _NUMRHO), jnp.float32),
        scratch_shapes=[pltpu.VMEM((_C, _P), jnp.bfloat16)],
    )(r3, v2)


def _asm_body(tc_ref, sc_ref, out_ref):
    out_ref[:, : _N_TC, :] = tc_ref[...]
    ii = jax.lax.broadcasted_iota(jnp.int32, (_NUMRHO, _NUMRHO), 0)
    jj = jax.lax.broadcasted_iota(jnp.int32, (_NUMRHO, _NUMRHO), 1)
    ident = jnp.where(ii == jj, 1.0, 0.0).astype(jnp.float32)
    for core in range(_NSC):
        for a in range(_APC):
            slab = sc_ref[core, pl.ds(a * _RSTRIDE, _NUMRHO), :]
            t = jax.lax.dot_general(
                slab, ident, (((0,), (0,)), ((), ())),
                preferred_element_type=jnp.float32,
            )
            out_ref[:, _N_TC + core * _APC + a, :] = t


def _asm_call(out_tc, out_sc):
    return pl.pallas_call(
        _asm_body,
        in_specs=[
            pl.BlockSpec((_C, _N_TC, _NUMRHO), lambda: (0, 0, 0)),
            pl.BlockSpec((_NSC, _RPAD, _C), lambda: (0, 0, 0)),
        ],
        out_specs=pl.BlockSpec((_C, _NUMANGLE, _NUMRHO), lambda: (0, 0, 0)),
        out_shape=jax.ShapeDtypeStruct((_C, _NUMANGLE, _NUMRHO), jnp.float32),
    )(out_tc, out_sc)


def kernel(x):
    v2 = x.reshape(_C, _P)

    vt = jnp.pad(jnp.maximum(v2, 0.0).T, ((0, _PPAD - _P), (0, 0)))
    out_sc = _dht_sc(vt, jnp.asarray(_IDXTAB))

    r3 = jnp.asarray(_RTAB[:_N_TC].astype(np.int16))[:, None, :]
    out_tc = _tc_call(r3, v2)

    out = _asm_call(out_tc, out_sc)
    return out.reshape(_B, _C, _NUMANGLE, _NUMRHO)
